# baseline (speedup 1.0000x reference)
import functools

import jax
import jax.numpy as jnp
from jax import lax
from jax.experimental import pallas as pl
from jax.experimental.pallas import tpu as pltpu
from jax.experimental.pallas import tpu_sc as plsc

NUM_CORES = 2
NUM_SUBCORES = 16
NUM_WORKERS = NUM_CORES * NUM_SUBCORES

B = 4096 * 26
D = 64
B_PER_W = B // NUM_WORKERS
KFIRE = 16


@jax.jit
def _sc_gather(weight, idx_flat):
    mesh = plsc.VectorSubcoreMesh(core_axis_name="c", subcore_axis_name="s")

    @functools.partial(
        pl.kernel,
        mesh=mesh,
        out_type=jax.ShapeDtypeStruct((B, D), jnp.float32),
        scratch_types=[
            pltpu.VMEM((B_PER_W,), jnp.int32),
            pltpu.SemaphoreType.DMA,
        ],
    )
    def k(table_hbm, idx_hbm, out_hbm, idx_s, sem):
        wid = lax.axis_index("s") * NUM_CORES + lax.axis_index("c")
        base = wid * B_PER_W
        pltpu.sync_copy(idx_hbm.at[pl.ds(base, B_PER_W)], idx_s)

        @pl.loop(0, B_PER_W, step=KFIRE)
        def _(i):
            vec = idx_s[pl.ds(i, KFIRE)]
            copies = []
            for j in range(KFIRE):
                idx = vec[j]
                copies.append(
                    pltpu.async_copy(
                        table_hbm.at[idx], out_hbm.at[base + i + j], sem
                    )
                )
            for c in copies:
                c.wait()

    return k(weight, idx_flat)


def kernel(x, weight):
    s = x.shape
    idx_flat = x.reshape(-1).astype(jnp.int32)
    out = _sc_gather(weight, idx_flat)
    return out.reshape(s + (weight.shape[1],))


# speedup vs baseline: 2.5522x; 2.5522x over previous
import functools

import jax
import jax.numpy as jnp
from jax import lax
from jax.experimental import pallas as pl
from jax.experimental.pallas import tpu as pltpu
from jax.experimental.pallas import tpu_sc as plsc

NUM_CORES = 2
NUM_SUBCORES = 16
NUM_WORKERS = NUM_CORES * NUM_SUBCORES

B = 4096 * 26
D = 64
B_PER_W = B // NUM_WORKERS
CHUNK = 416
NCHUNK = B_PER_W // CHUNK


@jax.jit
def _sc_gather_pairs(w2, idx2):
    mesh = plsc.VectorSubcoreMesh(core_axis_name="c", subcore_axis_name="s")

    @functools.partial(
        pl.kernel,
        mesh=mesh,
        out_type=jax.ShapeDtypeStruct((B, 2 * D), jnp.float32),
        scratch_types=[
            pltpu.VMEM((CHUNK,), jnp.int32),
            pltpu.VMEM((CHUNK, 2 * D), jnp.float32),
            pltpu.SemaphoreType.DMA,
        ],
    )
    def k(table_hbm, idx_hbm, out_hbm, idx_v, rows_v, sem):
        wid = lax.axis_index("s") * NUM_CORES + lax.axis_index("c")
        base = wid * B_PER_W
        for c in range(NCHUNK):
            off = base + c * CHUNK
            pltpu.sync_copy(idx_hbm.at[pl.ds(off, CHUNK)], idx_v)
            pltpu.async_copy(table_hbm.at[idx_v], rows_v, sem).wait()
            pltpu.sync_copy(rows_v, out_hbm.at[pl.ds(off, CHUNK)])

    return k(w2, idx2)


def kernel(x, weight):
    s = x.shape
    idx_flat = x.reshape(-1).astype(jnp.int32)
    w2 = weight.reshape(weight.shape[0] // 2, 2 * D)
    pairs = _sc_gather_pairs(w2, idx_flat >> 1)
    parity = (idx_flat & 1).astype(bool)
    out = jnp.where(parity[:, None], pairs[:, D:], pairs[:, :D])
    return out.reshape(s + (weight.shape[1],))
